# Initial kernel scaffold; baseline (speedup 1.0000x reference)
import jax
import jax.numpy as jnp
from jax.experimental import pallas as pl


def kernel(x_lin, x_conv, edge_index, W_lin, b_lin, W_gat, att_src, att_dst, b_gat):
    raise NotImplementedError("write your pallas kernel here")


# speedup vs baseline: 19.6225x; 19.6225x over previous
import functools

import jax
import jax.numpy as jnp
from jax import lax
from jax.experimental import pallas as pl
from jax.experimental.pallas import tpu as pltpu
from jax.experimental.pallas import tpu_sc as plsc

N = 10000
E = 320000
D = 128
O = 128

NC = 2
NS = 16
NW = NC * NS
B = 128
CPT = 81
EPT = B * CPT
EPAD = NW * EPT
ACC_ROWS = 10240
RPT = ACC_ROWS // NS

BLK = 512
GRID = ACC_ROWS // BLK


def _tc1_body(xl_ref, xc_ref, wg1_ref, wg2_ref, asv_ref, adv_ref,
              h_ref, aa_ref):
    h = jnp.dot(xl_ref[...], wg1_ref[...], preferred_element_type=jnp.float32)
    h += jnp.dot(xc_ref[...], wg2_ref[...], preferred_element_type=jnp.float32)
    h_ref[...] = h
    aa_ref[0, :] = jnp.dot(h, asv_ref[0, :], preferred_element_type=jnp.float32)
    aa_ref[1, :] = jnp.dot(h, adv_ref[0, :], preferred_element_type=jnp.float32)


def _tc1(x_lin, x_conv, wg1, wg2, att_src2, att_dst2):
    return pl.pallas_call(
        _tc1_body,
        grid=(GRID,),
        in_specs=[
            pl.BlockSpec((BLK, D), lambda i: (i, 0)),
            pl.BlockSpec((BLK, D), lambda i: (i, 0)),
            pl.BlockSpec((D, O), lambda i: (0, 0)),
            pl.BlockSpec((D, O), lambda i: (0, 0)),
            pl.BlockSpec((1, O), lambda i: (0, 0)),
            pl.BlockSpec((1, O), lambda i: (0, 0)),
        ],
        out_specs=[
            pl.BlockSpec((BLK, O), lambda i: (i, 0)),
            pl.BlockSpec((2, BLK), lambda i: (0, i)),
        ],
        out_shape=[
            jax.ShapeDtypeStruct((N, O), jnp.float32),
            jax.ShapeDtypeStruct((2, ACC_ROWS), jnp.float32),
        ],
    )(x_lin, x_conv, wg1, wg2, att_src2, att_dst2)


def _sc_body(h_hbm, aa_hbm, src_hbm, dst_hbm,
             acc_out, den_out,
             sidx_c, didx_c, rows, evals, asrc_v, adst_v, den_v,
             acc_sh, sem):
    c = lax.axis_index("c")
    s = lax.axis_index("s")
    wid = s * NC + c

    pltpu.sync_copy(aa_hbm.at[0], asrc_v)
    pltpu.sync_copy(aa_hbm.at[1], adst_v)

    zer = jnp.zeros((16,), jnp.float32)

    def zero_den(i, _):
        den_v[pl.ds(i * 16, 16)] = zer
        return 0
    lax.fori_loop(0, ACC_ROWS // 16, zero_den, 0, unroll=8)

    def zero_rows(i, _):
        for k in range(8):
            rows[i, pl.ds(k * 16, 16)] = zer
        return 0
    lax.fori_loop(0, B, zero_rows, 0, unroll=4)

    for k in range(RPT // B):
        pltpu.sync_copy(rows, acc_sh.at[pl.ds(s * RPT + k * B, B)])

    plsc.subcore_barrier()

    def chunk(g, _):
        pltpu.sync_copy(src_hbm.at[wid, g], sidx_c)
        pltpu.sync_copy(dst_hbm.at[wid, g], didx_c)
        pltpu.async_copy(h_hbm.at[sidx_c], rows, sem).wait()
        for j in range(B // 16):
            s16 = sidx_c[pl.ds(j * 16, 16)]
            d16 = didx_c[pl.ds(j * 16, 16)]
            a_s = plsc.load_gather(asrc_v, [s16])
            a_d = plsc.load_gather(adst_v, [d16])
            al = a_s + a_d
            al = jnp.where(al >= 0.0, al, al * 0.2)
            e = jnp.exp(al)
            plsc.addupdate_scatter(den_v, [d16], e)
            evals[pl.ds(j * 16, 16)] = e

        def scale(i, _):
            eb = plsc.load_gather(evals, [jnp.full((16,), 0, jnp.int32) + i])
            for k in range(8):
                rows[i, pl.ds(k * 16, 16)] = rows[i, pl.ds(k * 16, 16)] * eb
            return 0
        lax.fori_loop(0, B, scale, 0, unroll=2)

        pltpu.sync_copy(rows, acc_sh.at[didx_c], add=True)
        return 0

    lax.fori_loop(0, CPT, chunk, 0)

    pltpu.sync_copy(den_v, den_out.at[wid])

    plsc.subcore_barrier()
    pltpu.sync_copy(acc_sh.at[pl.ds(s * RPT, RPT)],
                    acc_out.at[c, pl.ds(s * RPT, RPT)])


def _sc_call(h, aa, src_t, dst_t):
    mesh = plsc.VectorSubcoreMesh(core_axis_name="c", subcore_axis_name="s",
                                  num_cores=NC, num_subcores=NS)
    f = pl.kernel(
        _sc_body,
        out_type=[
            jax.ShapeDtypeStruct((NC, ACC_ROWS, O), jnp.float32),
            jax.ShapeDtypeStruct((NW, ACC_ROWS), jnp.float32),
        ],
        mesh=mesh,
        compiler_params=pltpu.CompilerParams(needs_layout_passes=False),
        scratch_types=[
            pltpu.VMEM((B,), jnp.int32),
            pltpu.VMEM((B,), jnp.int32),
            pltpu.VMEM((B, O), jnp.float32),
            pltpu.VMEM((B,), jnp.float32),
            pltpu.VMEM((ACC_ROWS,), jnp.float32),
            pltpu.VMEM((ACC_ROWS,), jnp.float32),
            pltpu.VMEM((ACC_ROWS,), jnp.float32),
            pltpu.VMEM_SHARED((ACC_ROWS, O), jnp.float32),
            pltpu.SemaphoreType.DMA,
        ],
    )
    return f(h, aa, src_t, dst_t)


def _tc2_body(acc_ref, den_ref, xl_ref, wlt_ref, bl_ref, bg_ref,
              ol_ref, oc_ref):
    ssum = acc_ref[0] + acc_ref[1]
    d = jnp.sum(den_ref[...], axis=0)
    coef = 1.0 / (d + 1e-16)
    oc_ref[...] = ssum * coef[:, None] + bg_ref[0, :]
    ol_ref[...] = jnp.dot(xl_ref[...], wlt_ref[...],
                          preferred_element_type=jnp.float32) + bl_ref[0, :]


def _tc2(acc, den, x_lin, wlt, b_lin2, b_gat2):
    return pl.pallas_call(
        _tc2_body,
        grid=(GRID,),
        in_specs=[
            pl.BlockSpec((2, BLK, O), lambda i: (0, i, 0)),
            pl.BlockSpec((NW, BLK), lambda i: (0, i)),
            pl.BlockSpec((BLK, D), lambda i: (i, 0)),
            pl.BlockSpec((D, O), lambda i: (0, 0)),
            pl.BlockSpec((1, O), lambda i: (0, 0)),
            pl.BlockSpec((1, O), lambda i: (0, 0)),
        ],
        out_specs=[
            pl.BlockSpec((BLK, O), lambda i: (i, 0)),
            pl.BlockSpec((BLK, O), lambda i: (i, 0)),
        ],
        out_shape=[
            jax.ShapeDtypeStruct((N, O), jnp.float32),
            jax.ShapeDtypeStruct((N, O), jnp.float32),
        ],
    )(acc, den, x_lin, wlt, b_lin2, b_gat2)


def kernel(x_lin, x_conv, edge_index, W_lin, b_lin, W_gat, att_src, att_dst,
           b_gat):
    ei = edge_index.astype(jnp.int32)
    loop = jnp.arange(N, dtype=jnp.int32)
    src = jnp.concatenate([ei[0], loop,
                           jnp.zeros((EPAD - E - N,), jnp.int32)])
    dst = jnp.concatenate([ei[1], loop,
                           jnp.full((EPAD - E - N,), N, jnp.int32)])
    src_t = src.reshape(NW, CPT, B)
    dst_t = dst.reshape(NW, CPT, B)

    wg1 = W_gat[:, :D].T
    wg2 = W_gat[:, D:].T
    att_src2 = att_src.reshape(1, O)
    att_dst2 = att_dst.reshape(1, O)

    h, aa = _tc1(x_lin, x_conv, wg1, wg2, att_src2, att_dst2)

    acc, den = _sc_call(h, aa, src_t, dst_t)

    wlt = W_lin.T
    out_lin, out_conv = _tc2(acc, den, x_lin, wlt,
                             b_lin.reshape(1, O), b_gat.reshape(1, O))
    return (out_lin, out_conv)


# speedup vs baseline: 25.0913x; 1.2787x over previous
import functools

import jax
import jax.numpy as jnp
from jax import lax
from jax.experimental import pallas as pl
from jax.experimental.pallas import tpu as pltpu
from jax.experimental.pallas import tpu_sc as plsc

N = 10000
E = 320000
D = 128
O = 128

NC = 2
NS = 16
NW = NC * NS
B = 64
CPT = 162
EPT = B * CPT
EPAD = NW * EPT
ACC_ROWS = 10240
RPT = ACC_ROWS // NS

BLK = 512
GRID = ACC_ROWS // BLK


def _tc1_body(xl_ref, xc_ref, wg1_ref, wg2_ref, asv_ref, adv_ref,
              h_ref, aa_ref):
    h = jnp.dot(xl_ref[...], wg1_ref[...], preferred_element_type=jnp.float32)
    h += jnp.dot(xc_ref[...], wg2_ref[...], preferred_element_type=jnp.float32)
    h_ref[...] = h
    aa_ref[0, :] = jnp.dot(h, asv_ref[0, :], preferred_element_type=jnp.float32)
    aa_ref[1, :] = jnp.dot(h, adv_ref[0, :], preferred_element_type=jnp.float32)


def _tc1(x_lin, x_conv, wg1, wg2, att_src2, att_dst2):
    return pl.pallas_call(
        _tc1_body,
        grid=(GRID,),
        in_specs=[
            pl.BlockSpec((BLK, D), lambda i: (i, 0)),
            pl.BlockSpec((BLK, D), lambda i: (i, 0)),
            pl.BlockSpec((D, O), lambda i: (0, 0)),
            pl.BlockSpec((D, O), lambda i: (0, 0)),
            pl.BlockSpec((1, O), lambda i: (0, 0)),
            pl.BlockSpec((1, O), lambda i: (0, 0)),
        ],
        out_specs=[
            pl.BlockSpec((BLK, O), lambda i: (i, 0)),
            pl.BlockSpec((2, BLK), lambda i: (0, i)),
        ],
        out_shape=[
            jax.ShapeDtypeStruct((N, O), jnp.float32),
            jax.ShapeDtypeStruct((2, ACC_ROWS), jnp.float32),
        ],
    )(x_lin, x_conv, wg1, wg2, att_src2, att_dst2)


def _sc_body(h_hbm, aa_hbm, src_hbm, dst_hbm,
             acc_out, den_out,
             sidx0, sidx1, didx0, didx1, rows0, rows1, evals,
             asrc_v, adst_v, den_v, acc_sh,
             sem_r0, sem_r1, sem_i0, sem_i1):
    c = lax.axis_index("c")
    s = lax.axis_index("s")
    wid = s * NC + c
    sidx = (sidx0, sidx1)
    didx = (didx0, didx1)
    rows = (rows0, rows1)
    sem_r = (sem_r0, sem_r1)
    sem_i = (sem_i0, sem_i1)

    pltpu.sync_copy(aa_hbm.at[0], asrc_v)
    pltpu.sync_copy(aa_hbm.at[1], adst_v)

    zer = jnp.zeros((16,), jnp.float32)

    def zero_den(i, _):
        den_v[pl.ds(i * 16, 16)] = zer
        return 0
    lax.fori_loop(0, ACC_ROWS // 16, zero_den, 0, unroll=8)

    def zero_rows(i, _):
        for k in range(8):
            rows0[i, pl.ds(k * 16, 16)] = zer
            rows1[i, pl.ds(k * 16, 16)] = zer
        return 0
    lax.fori_loop(0, B, zero_rows, 0, unroll=4)

    for k in range(RPT // B):
        pltpu.sync_copy(rows0, acc_sh.at[pl.ds(s * RPT + k * B, B)])

    plsc.subcore_barrier()

    pltpu.sync_copy(src_hbm.at[wid, 0], sidx0)
    pltpu.sync_copy(dst_hbm.at[wid, 0], didx0)
    pltpu.async_copy(h_hbm.at[sidx0], rows0, sem_r0)
    pltpu.async_copy(src_hbm.at[wid, 1], sidx1, sem_i1)
    pltpu.async_copy(dst_hbm.at[wid, 1], didx1, sem_i1)

    def outer(i, _):
        for b in range(2):
            g = i * 2 + b
            nb = 1 - b

            @pl.when(g + 1 < CPT)
            def _():
                pltpu.make_async_copy(src_hbm.at[wid, 0], sidx[nb],
                                      sem_i[nb]).wait()
                pltpu.make_async_copy(dst_hbm.at[wid, 0], didx[nb],
                                      sem_i[nb]).wait()
                pltpu.async_copy(h_hbm.at[sidx[nb]], rows[nb], sem_r[nb])

            pltpu.make_async_copy(h_hbm.at[sidx[b]], rows[b], sem_r[b]).wait()

            for j in range(B // 16):
                s16 = sidx[b][pl.ds(j * 16, 16)]
                d16 = didx[b][pl.ds(j * 16, 16)]
                a_s = plsc.load_gather(asrc_v, [s16])
                a_d = plsc.load_gather(adst_v, [d16])
                al = a_s + a_d
                al = jnp.where(al >= 0.0, al, al * 0.2)
                e = jnp.exp(al)
                plsc.addupdate_scatter(den_v, [d16], e)
                evals[pl.ds(j * 16, 16)] = e

            def scale(k, _):
                eb = plsc.load_gather(evals, [jnp.full((16,), 0, jnp.int32) + k])
                for q in range(8):
                    rows[b][k, pl.ds(q * 16, 16)] = (
                        rows[b][k, pl.ds(q * 16, 16)] * eb)
                return 0
            lax.fori_loop(0, B, scale, 0, unroll=2)

            pltpu.sync_copy(rows[b], acc_sh.at[didx[b]], add=True)

            @pl.when(g + 2 < CPT)
            def _():
                pltpu.async_copy(src_hbm.at[wid, g + 2], sidx[b], sem_i[b])
                pltpu.async_copy(dst_hbm.at[wid, g + 2], didx[b], sem_i[b])
        return 0

    lax.fori_loop(0, CPT // 2, outer, 0)

    pltpu.sync_copy(den_v, den_out.at[wid])

    plsc.subcore_barrier()
    pltpu.sync_copy(acc_sh.at[pl.ds(s * RPT, RPT)],
                    acc_out.at[c, pl.ds(s * RPT, RPT)])


def _sc_call(h, aa, src_t, dst_t):
    mesh = plsc.VectorSubcoreMesh(core_axis_name="c", subcore_axis_name="s",
                                  num_cores=NC, num_subcores=NS)
    f = pl.kernel(
        _sc_body,
        out_type=[
            jax.ShapeDtypeStruct((NC, ACC_ROWS, O), jnp.float32),
            jax.ShapeDtypeStruct((NW, ACC_ROWS), jnp.float32),
        ],
        mesh=mesh,
        compiler_params=pltpu.CompilerParams(needs_layout_passes=False),
        scratch_types=[
            pltpu.VMEM((B,), jnp.int32),
            pltpu.VMEM((B,), jnp.int32),
            pltpu.VMEM((B,), jnp.int32),
            pltpu.VMEM((B,), jnp.int32),
            pltpu.VMEM((B, O), jnp.float32),
            pltpu.VMEM((B, O), jnp.float32),
            pltpu.VMEM((B,), jnp.float32),
            pltpu.VMEM((ACC_ROWS,), jnp.float32),
            pltpu.VMEM((ACC_ROWS,), jnp.float32),
            pltpu.VMEM((ACC_ROWS,), jnp.float32),
            pltpu.VMEM_SHARED((ACC_ROWS, O), jnp.float32),
            pltpu.SemaphoreType.DMA,
            pltpu.SemaphoreType.DMA,
            pltpu.SemaphoreType.DMA,
            pltpu.SemaphoreType.DMA,
        ],
    )
    return f(h, aa, src_t, dst_t)


def _tc2_body(acc_ref, den_ref, xl_ref, wlt_ref, bl_ref, bg_ref,
              ol_ref, oc_ref):
    ssum = acc_ref[0] + acc_ref[1]
    d = jnp.sum(den_ref[...], axis=0)
    coef = 1.0 / (d + 1e-16)
    oc_ref[...] = ssum * coef[:, None] + bg_ref[0, :]
    ol_ref[...] = jnp.dot(xl_ref[...], wlt_ref[...],
                          preferred_element_type=jnp.float32) + bl_ref[0, :]


def _tc2(acc, den, x_lin, wlt, b_lin2, b_gat2):
    return pl.pallas_call(
        _tc2_body,
        grid=(GRID,),
        in_specs=[
            pl.BlockSpec((2, BLK, O), lambda i: (0, i, 0)),
            pl.BlockSpec((NW, BLK), lambda i: (0, i)),
            pl.BlockSpec((BLK, D), lambda i: (i, 0)),
            pl.BlockSpec((D, O), lambda i: (0, 0)),
            pl.BlockSpec((1, O), lambda i: (0, 0)),
            pl.BlockSpec((1, O), lambda i: (0, 0)),
        ],
        out_specs=[
            pl.BlockSpec((BLK, O), lambda i: (i, 0)),
            pl.BlockSpec((BLK, O), lambda i: (i, 0)),
        ],
        out_shape=[
            jax.ShapeDtypeStruct((N, O), jnp.float32),
            jax.ShapeDtypeStruct((N, O), jnp.float32),
        ],
    )(acc, den, x_lin, wlt, b_lin2, b_gat2)


def kernel(x_lin, x_conv, edge_index, W_lin, b_lin, W_gat, att_src, att_dst,
           b_gat):
    ei = edge_index.astype(jnp.int32)
    loop = jnp.arange(N, dtype=jnp.int32)
    src = jnp.concatenate([ei[0], loop,
                           jnp.zeros((EPAD - E - N,), jnp.int32)])
    dst = jnp.concatenate([ei[1], loop,
                           jnp.full((EPAD - E - N,), N, jnp.int32)])
    src_t = src.reshape(NW, CPT, B)
    dst_t = dst.reshape(NW, CPT, B)

    wg1 = W_gat[:, :D].T
    wg2 = W_gat[:, D:].T
    att_src2 = att_src.reshape(1, O)
    att_dst2 = att_dst.reshape(1, O)

    h, aa = _tc1(x_lin, x_conv, wg1, wg2, att_src2, att_dst2)

    acc, den = _sc_call(h, aa, src_t, dst_t)

    wlt = W_lin.T
    out_lin, out_conv = _tc2(acc, den, x_lin, wlt,
                             b_lin.reshape(1, O), b_gat.reshape(1, O))
    return (out_lin, out_conv)


# speedup vs baseline: 28.3662x; 1.1305x over previous
import functools

import jax
import jax.numpy as jnp
from jax import lax
from jax.experimental import pallas as pl
from jax.experimental.pallas import tpu as pltpu
from jax.experimental.pallas import tpu_sc as plsc

N = 10000
E = 320000
D = 128
O = 128

NC = 2
NS = 16
NW = NC * NS
B = 64
CPT = 162
EPT = B * CPT
EPAD = NW * EPT
ACC_ROWS = 10240
RPT = ACC_ROWS // NS

BLK = 512
GRID = ACC_ROWS // BLK


def _tc1_body(xl_ref, xc_ref, wg1_ref, wg2_ref, asv_ref, adv_ref,
              h_ref, aa_ref):
    h = jnp.dot(xl_ref[...], wg1_ref[...], preferred_element_type=jnp.float32)
    h += jnp.dot(xc_ref[...], wg2_ref[...], preferred_element_type=jnp.float32)
    h_ref[...] = h
    aa_ref[0, :] = jnp.dot(h, asv_ref[0, :], preferred_element_type=jnp.float32)
    aa_ref[1, :] = jnp.dot(h, adv_ref[0, :], preferred_element_type=jnp.float32)


def _tc1(x_lin, x_conv, wg1, wg2, att_src2, att_dst2):
    return pl.pallas_call(
        _tc1_body,
        grid=(GRID,),
        in_specs=[
            pl.BlockSpec((BLK, D), lambda i: (i, 0)),
            pl.BlockSpec((BLK, D), lambda i: (i, 0)),
            pl.BlockSpec((D, O), lambda i: (0, 0)),
            pl.BlockSpec((D, O), lambda i: (0, 0)),
            pl.BlockSpec((1, O), lambda i: (0, 0)),
            pl.BlockSpec((1, O), lambda i: (0, 0)),
        ],
        out_specs=[
            pl.BlockSpec((BLK, O), lambda i: (i, 0)),
            pl.BlockSpec((2, BLK), lambda i: (0, i)),
        ],
        out_shape=[
            jax.ShapeDtypeStruct((N, O), jnp.float32),
            jax.ShapeDtypeStruct((2, ACC_ROWS), jnp.float32),
        ],
    )(x_lin, x_conv, wg1, wg2, att_src2, att_dst2)


def _sc_body(h_hbm, aa_hbm, src_hbm, dst_hbm,
             acc_out, den_out,
             sidx0, sidx1, didx0, didx1, cidx0, cidx1, rows0, rows1, evals,
             asrc_v, adst_v, den_v, acc_sh,
             sem_r0, sem_r1, sem_i0, sem_i1, sem_s0, sem_s1):
    c = lax.axis_index("c")
    s = lax.axis_index("s")
    wid = s * NC + c
    sidx = (sidx0, sidx1)
    didx = (didx0, didx1)
    cidx = (cidx0, cidx1)
    rows = (rows0, rows1)
    sem_r = (sem_r0, sem_r1)
    sem_i = (sem_i0, sem_i1)
    sem_s = (sem_s0, sem_s1)

    pltpu.sync_copy(aa_hbm.at[0], asrc_v)
    pltpu.sync_copy(aa_hbm.at[1], adst_v)

    zer = jnp.zeros((16,), jnp.float32)

    def zero_den(i, _):
        den_v[pl.ds(i * 16, 16)] = zer
        return 0
    lax.fori_loop(0, ACC_ROWS // 16, zero_den, 0, unroll=8)

    def zero_rows(i, _):
        for k in range(8):
            rows0[i, pl.ds(k * 16, 16)] = zer
            rows1[i, pl.ds(k * 16, 16)] = zer
        return 0
    lax.fori_loop(0, B, zero_rows, 0, unroll=4)

    for k in range(RPT // B):
        pltpu.sync_copy(rows0, acc_sh.at[pl.ds(s * RPT + k * B, B)])

    plsc.subcore_barrier()

    pltpu.sync_copy(src_hbm.at[wid, 0], sidx0)
    pltpu.sync_copy(dst_hbm.at[wid, 0], didx0)
    pltpu.async_copy(h_hbm.at[sidx0], rows0, sem_r0)
    pltpu.async_copy(src_hbm.at[wid, 1], sidx1, sem_i1)
    pltpu.async_copy(dst_hbm.at[wid, 1], didx1, sem_i1)

    def outer(i, _):
        for b in range(2):
            g = i * 2 + b
            nb = 1 - b

            @pl.when(jnp.logical_and(g + 1 < CPT, g >= 1))
            def _():
                pltpu.make_async_copy(rows[nb], acc_sh.at[cidx[nb]],
                                      sem_s[nb]).wait()

            @pl.when(g + 1 < CPT)
            def _():
                pltpu.make_async_copy(src_hbm.at[wid, 0], sidx[nb],
                                      sem_i[nb]).wait()
                pltpu.make_async_copy(dst_hbm.at[wid, 0], didx[nb],
                                      sem_i[nb]).wait()
                pltpu.async_copy(h_hbm.at[sidx[nb]], rows[nb], sem_r[nb])

            pltpu.make_async_copy(h_hbm.at[sidx[b]], rows[b], sem_r[b]).wait()

            for j in range(B // 16):
                s16 = sidx[b][pl.ds(j * 16, 16)]
                d16 = didx[b][pl.ds(j * 16, 16)]
                a_s = plsc.load_gather(asrc_v, [s16])
                a_d = plsc.load_gather(adst_v, [d16])
                al = a_s + a_d
                al = jnp.where(al >= 0.0, al, al * 0.2)
                e = jnp.exp(al)
                plsc.addupdate_scatter(den_v, [d16], e)
                evals[pl.ds(j * 16, 16)] = e

            def scale(k, _):
                eb = plsc.load_gather(evals, [jnp.full((16,), 0, jnp.int32) + k])
                for q in range(8):
                    rows[b][k, pl.ds(q * 16, 16)] = (
                        rows[b][k, pl.ds(q * 16, 16)] * eb)
                return 0
            lax.fori_loop(0, B, scale, 0, unroll=4)

            for q in range(B // 16):
                cidx[b][pl.ds(q * 16, 16)] = didx[b][pl.ds(q * 16, 16)]
            pltpu.async_copy(rows[b], acc_sh.at[cidx[b]], sem_s[b], add=True)

            @pl.when(g + 2 < CPT)
            def _():
                pltpu.async_copy(src_hbm.at[wid, g + 2], sidx[b], sem_i[b])
                pltpu.async_copy(dst_hbm.at[wid, g + 2], didx[b], sem_i[b])
        return 0

    lax.fori_loop(0, CPT // 2, outer, 0)

    pltpu.make_async_copy(rows[0], acc_sh.at[cidx[0]], sem_s[0]).wait()
    pltpu.make_async_copy(rows[1], acc_sh.at[cidx[1]], sem_s[1]).wait()

    pltpu.sync_copy(den_v, den_out.at[wid])

    plsc.subcore_barrier()
    pltpu.sync_copy(acc_sh.at[pl.ds(s * RPT, RPT)],
                    acc_out.at[c, pl.ds(s * RPT, RPT)])


def _sc_call(h, aa, src_t, dst_t):
    mesh = plsc.VectorSubcoreMesh(core_axis_name="c", subcore_axis_name="s",
                                  num_cores=NC, num_subcores=NS)
    f = pl.kernel(
        _sc_body,
        out_type=[
            jax.ShapeDtypeStruct((NC, ACC_ROWS, O), jnp.float32),
            jax.ShapeDtypeStruct((NW, ACC_ROWS), jnp.float32),
        ],
        mesh=mesh,
        compiler_params=pltpu.CompilerParams(needs_layout_passes=False),
        scratch_types=[
            pltpu.VMEM((B,), jnp.int32),
            pltpu.VMEM((B,), jnp.int32),
            pltpu.VMEM((B,), jnp.int32),
            pltpu.VMEM((B,), jnp.int32),
            pltpu.VMEM((B,), jnp.int32),
            pltpu.VMEM((B,), jnp.int32),
            pltpu.VMEM((B, O), jnp.float32),
            pltpu.VMEM((B, O), jnp.float32),
            pltpu.VMEM((B,), jnp.float32),
            pltpu.VMEM((ACC_ROWS,), jnp.float32),
            pltpu.VMEM((ACC_ROWS,), jnp.float32),
            pltpu.VMEM((ACC_ROWS,), jnp.float32),
            pltpu.VMEM_SHARED((ACC_ROWS, O), jnp.float32),
            pltpu.SemaphoreType.DMA,
            pltpu.SemaphoreType.DMA,
            pltpu.SemaphoreType.DMA,
            pltpu.SemaphoreType.DMA,
            pltpu.SemaphoreType.DMA,
            pltpu.SemaphoreType.DMA,
        ],
    )
    return f(h, aa, src_t, dst_t)


def _tc2_body(acc_ref, den_ref, xl_ref, wlt_ref, bl_ref, bg_ref,
              ol_ref, oc_ref):
    ssum = acc_ref[0] + acc_ref[1]
    d = jnp.sum(den_ref[...], axis=0)
    coef = 1.0 / (d + 1e-16)
    oc_ref[...] = ssum * coef[:, None] + bg_ref[0, :]
    ol_ref[...] = jnp.dot(xl_ref[...], wlt_ref[...],
                          preferred_element_type=jnp.float32) + bl_ref[0, :]


def _tc2(acc, den, x_lin, wlt, b_lin2, b_gat2):
    return pl.pallas_call(
        _tc2_body,
        grid=(GRID,),
        in_specs=[
            pl.BlockSpec((2, BLK, O), lambda i: (0, i, 0)),
            pl.BlockSpec((NW, BLK), lambda i: (0, i)),
            pl.BlockSpec((BLK, D), lambda i: (i, 0)),
            pl.BlockSpec((D, O), lambda i: (0, 0)),
            pl.BlockSpec((1, O), lambda i: (0, 0)),
            pl.BlockSpec((1, O), lambda i: (0, 0)),
        ],
        out_specs=[
            pl.BlockSpec((BLK, O), lambda i: (i, 0)),
            pl.BlockSpec((BLK, O), lambda i: (i, 0)),
        ],
        out_shape=[
            jax.ShapeDtypeStruct((N, O), jnp.float32),
            jax.ShapeDtypeStruct((N, O), jnp.float32),
        ],
    )(acc, den, x_lin, wlt, b_lin2, b_gat2)


def kernel(x_lin, x_conv, edge_index, W_lin, b_lin, W_gat, att_src, att_dst,
           b_gat):
    ei = edge_index.astype(jnp.int32)
    loop = jnp.arange(N, dtype=jnp.int32)
    src = jnp.concatenate([ei[0], loop,
                           jnp.zeros((EPAD - E - N,), jnp.int32)])
    dst = jnp.concatenate([ei[1], loop,
                           jnp.full((EPAD - E - N,), N, jnp.int32)])
    src_t = src.reshape(NW, CPT, B)
    dst_t = dst.reshape(NW, CPT, B)

    wg1 = W_gat[:, :D].T
    wg2 = W_gat[:, D:].T
    att_src2 = att_src.reshape(1, O)
    att_dst2 = att_dst.reshape(1, O)

    h, aa = _tc1(x_lin, x_conv, wg1, wg2, att_src2, att_dst2)

    acc, den = _sc_call(h, aa, src_t, dst_t)

    wlt = W_lin.T
    out_lin, out_conv = _tc2(acc, den, x_lin, wlt,
                             b_lin.reshape(1, O), b_gat.reshape(1, O))
    return (out_lin, out_conv)


# speedup vs baseline: 29.6539x; 1.0454x over previous
import functools

import jax
import jax.numpy as jnp
from jax import lax
from jax.experimental import pallas as pl
from jax.experimental.pallas import tpu as pltpu
from jax.experimental.pallas import tpu_sc as plsc

N = 10000
E = 320000
D = 128
O = 128

NC = 2
NS = 16
NW = NC * NS
B = 96
CPT = 108
EPT = B * CPT
EPAD = NW * EPT
ACC_ROWS = 10240
RPT = ACC_ROWS // NS

BLK = 512
GRID = ACC_ROWS // BLK


def _tc1_body(xl_ref, xc_ref, wg1_ref, wg2_ref, asv_ref, adv_ref,
              h_ref, aa_ref):
    h = jnp.dot(xl_ref[...], wg1_ref[...], preferred_element_type=jnp.float32)
    h += jnp.dot(xc_ref[...], wg2_ref[...], preferred_element_type=jnp.float32)
    h_ref[...] = h
    aa_ref[0, :] = jnp.dot(h, asv_ref[0, :], preferred_element_type=jnp.float32)
    aa_ref[1, :] = jnp.dot(h, adv_ref[0, :], preferred_element_type=jnp.float32)


def _tc1(x_lin, x_conv, wg1, wg2, att_src2, att_dst2):
    return pl.pallas_call(
        _tc1_body,
        grid=(GRID,),
        in_specs=[
            pl.BlockSpec((BLK, D), lambda i: (i, 0)),
            pl.BlockSpec((BLK, D), lambda i: (i, 0)),
            pl.BlockSpec((D, O), lambda i: (0, 0)),
            pl.BlockSpec((D, O), lambda i: (0, 0)),
            pl.BlockSpec((1, O), lambda i: (0, 0)),
            pl.BlockSpec((1, O), lambda i: (0, 0)),
        ],
        out_specs=[
            pl.BlockSpec((BLK, O), lambda i: (i, 0)),
            pl.BlockSpec((2, BLK), lambda i: (0, i)),
        ],
        out_shape=[
            jax.ShapeDtypeStruct((N, O), jnp.float32),
            jax.ShapeDtypeStruct((2, ACC_ROWS), jnp.float32),
        ],
    )(x_lin, x_conv, wg1, wg2, att_src2, att_dst2)


def _sc_body(h_hbm, aa_hbm, src_hbm, dst_hbm,
             acc_out, den_out,
             sidx0, sidx1, didx0, didx1, cidx0, cidx1, rows0, rows1,
             evals0, evals1, asrc_v, adst_v, zbuf, acc_sh, den_sh,
             sem_r0, sem_r1, sem_i0, sem_i1, sem_s0, sem_s1):
    c = lax.axis_index("c")
    s = lax.axis_index("s")
    wid = s * NC + c
    sidx = (sidx0, sidx1)
    didx = (didx0, didx1)
    cidx = (cidx0, cidx1)
    rows = (rows0, rows1)
    evals = (evals0, evals1)
    sem_r = (sem_r0, sem_r1)
    sem_i = (sem_i0, sem_i1)
    sem_s = (sem_s0, sem_s1)

    pltpu.sync_copy(aa_hbm.at[0], asrc_v)
    pltpu.sync_copy(aa_hbm.at[1], adst_v)

    zer = jnp.zeros((16,), jnp.float32)

    def zero_zb(i, _):
        zbuf[pl.ds(i * 16, 16)] = zer
        return 0
    lax.fori_loop(0, RPT // 16, zero_zb, 0, unroll=8)

    def zero_rows(i, _):
        for k in range(8):
            rows0[i, pl.ds(k * 16, 16)] = zer
        return 0
    lax.fori_loop(0, B, zero_rows, 0, unroll=4)

    nfull = RPT // B
    for k in range(nfull):
        pltpu.sync_copy(rows0, acc_sh.at[pl.ds(s * RPT + k * B, B)])
    if RPT % B:
        pltpu.sync_copy(rows0.at[pl.ds(0, RPT % B)],
                        acc_sh.at[pl.ds(s * RPT + nfull * B, RPT % B)])
    pltpu.sync_copy(zbuf, den_sh.at[pl.ds(s * RPT, RPT)])

    plsc.subcore_barrier()

    pltpu.sync_copy(src_hbm.at[wid, 0], sidx0)
    pltpu.sync_copy(dst_hbm.at[wid, 0], didx0)
    pltpu.async_copy(h_hbm.at[sidx0], rows0, sem_r0)
    pltpu.async_copy(src_hbm.at[wid, 1], sidx1, sem_i1)
    pltpu.async_copy(dst_hbm.at[wid, 1], didx1, sem_i1)

    def outer(i, _):
        for b in range(2):
            g = i * 2 + b
            nb = 1 - b

            @pl.when(jnp.logical_and(g + 1 < CPT, g >= 1))
            def _():
                pltpu.make_async_copy(rows[nb], acc_sh.at[cidx[nb]],
                                      sem_s[nb]).wait()
                pltpu.make_async_copy(evals[nb], den_sh.at[cidx[nb]],
                                      sem_s[nb]).wait()

            @pl.when(g + 1 < CPT)
            def _():
                pltpu.make_async_copy(src_hbm.at[wid, 0], sidx[nb],
                                      sem_i[nb]).wait()
                pltpu.make_async_copy(dst_hbm.at[wid, 0], didx[nb],
                                      sem_i[nb]).wait()
                pltpu.async_copy(h_hbm.at[sidx[nb]], rows[nb], sem_r[nb])

            pltpu.make_async_copy(h_hbm.at[sidx[b]], rows[b], sem_r[b]).wait()

            for j in range(B // 16):
                s16 = sidx[b][pl.ds(j * 16, 16)]
                d16 = didx[b][pl.ds(j * 16, 16)]
                a_s = plsc.load_gather(asrc_v, [s16])
                a_d = plsc.load_gather(adst_v, [d16])
                al = a_s + a_d
                al = jnp.where(al >= 0.0, al, al * 0.2)
                e = jnp.exp(al)
                evals[b][pl.ds(j * 16, 16)] = e

            def scale(k, _):
                eb = plsc.load_gather(evals[b], [jnp.full((16,), 0, jnp.int32) + k])
                for q in range(8):
                    rows[b][k, pl.ds(q * 16, 16)] = (
                        rows[b][k, pl.ds(q * 16, 16)] * eb)
                return 0
            lax.fori_loop(0, B, scale, 0, unroll=4)

            for q in range(B // 16):
                cidx[b][pl.ds(q * 16, 16)] = didx[b][pl.ds(q * 16, 16)]
            pltpu.async_copy(rows[b], acc_sh.at[cidx[b]], sem_s[b], add=True)
            pltpu.async_copy(evals[b], den_sh.at[cidx[b]], sem_s[b], add=True)

            @pl.when(g + 2 < CPT)
            def _():
                pltpu.async_copy(src_hbm.at[wid, g + 2], sidx[b], sem_i[b])
                pltpu.async_copy(dst_hbm.at[wid, g + 2], didx[b], sem_i[b])
        return 0

    lax.fori_loop(0, CPT // 2, outer, 0)

    for b in range(2):
        pltpu.make_async_copy(rows[b], acc_sh.at[cidx[b]], sem_s[b]).wait()
        pltpu.make_async_copy(evals[b], den_sh.at[cidx[b]], sem_s[b]).wait()

    plsc.subcore_barrier()
    pltpu.sync_copy(acc_sh.at[pl.ds(s * RPT, RPT)],
                    acc_out.at[c, pl.ds(s * RPT, RPT)])
    pltpu.sync_copy(den_sh.at[pl.ds(s * RPT, RPT)],
                    den_out.at[c, pl.ds(s * RPT, RPT)])


def _sc_call(h, aa, src_t, dst_t):
    mesh = plsc.VectorSubcoreMesh(core_axis_name="c", subcore_axis_name="s",
                                  num_cores=NC, num_subcores=NS)
    f = pl.kernel(
        _sc_body,
        out_type=[
            jax.ShapeDtypeStruct((NC, ACC_ROWS, O), jnp.float32),
            jax.ShapeDtypeStruct((NC, ACC_ROWS), jnp.float32),
        ],
        mesh=mesh,
        compiler_params=pltpu.CompilerParams(needs_layout_passes=False),
        scratch_types=[
            pltpu.VMEM((B,), jnp.int32),
            pltpu.VMEM((B,), jnp.int32),
            pltpu.VMEM((B,), jnp.int32),
            pltpu.VMEM((B,), jnp.int32),
            pltpu.VMEM((B,), jnp.int32),
            pltpu.VMEM((B,), jnp.int32),
            pltpu.VMEM((B, O), jnp.float32),
            pltpu.VMEM((B, O), jnp.float32),
            pltpu.VMEM((B,), jnp.float32),
            pltpu.VMEM((B,), jnp.float32),
            pltpu.VMEM((ACC_ROWS,), jnp.float32),
            pltpu.VMEM((ACC_ROWS,), jnp.float32),
            pltpu.VMEM((RPT,), jnp.float32),
            pltpu.VMEM_SHARED((ACC_ROWS, O), jnp.float32),
            pltpu.VMEM_SHARED((ACC_ROWS,), jnp.float32),
            pltpu.SemaphoreType.DMA,
            pltpu.SemaphoreType.DMA,
            pltpu.SemaphoreType.DMA,
            pltpu.SemaphoreType.DMA,
            pltpu.SemaphoreType.DMA,
            pltpu.SemaphoreType.DMA,
        ],
    )
    return f(h, aa, src_t, dst_t)


def _tc2_body(acc_ref, den_ref, xl_ref, wlt_ref, bl_ref, bg_ref,
              ol_ref, oc_ref):
    ssum = acc_ref[0] + acc_ref[1]
    d = jnp.sum(den_ref[...], axis=0)
    coef = 1.0 / (d + 1e-16)
    oc_ref[...] = ssum * coef[:, None] + bg_ref[0, :]
    ol_ref[...] = jnp.dot(xl_ref[...], wlt_ref[...],
                          preferred_element_type=jnp.float32) + bl_ref[0, :]


def _tc2(acc, den, x_lin, wlt, b_lin2, b_gat2):
    return pl.pallas_call(
        _tc2_body,
        grid=(GRID,),
        in_specs=[
            pl.BlockSpec((2, BLK, O), lambda i: (0, i, 0)),
            pl.BlockSpec((2, BLK), lambda i: (0, i)),
            pl.BlockSpec((BLK, D), lambda i: (i, 0)),
            pl.BlockSpec((D, O), lambda i: (0, 0)),
            pl.BlockSpec((1, O), lambda i: (0, 0)),
            pl.BlockSpec((1, O), lambda i: (0, 0)),
        ],
        out_specs=[
            pl.BlockSpec((BLK, O), lambda i: (i, 0)),
            pl.BlockSpec((BLK, O), lambda i: (i, 0)),
        ],
        out_shape=[
            jax.ShapeDtypeStruct((N, O), jnp.float32),
            jax.ShapeDtypeStruct((N, O), jnp.float32),
        ],
    )(acc, den, x_lin, wlt, b_lin2, b_gat2)


def kernel(x_lin, x_conv, edge_index, W_lin, b_lin, W_gat, att_src, att_dst,
           b_gat):
    ei = edge_index.astype(jnp.int32)
    loop = jnp.arange(N, dtype=jnp.int32)
    src = jnp.concatenate([ei[0], loop,
                           jnp.zeros((EPAD - E - N,), jnp.int32)])
    dst = jnp.concatenate([ei[1], loop,
                           jnp.full((EPAD - E - N,), N, jnp.int32)])
    src_t = src.reshape(NW, CPT, B)
    dst_t = dst.reshape(NW, CPT, B)

    wg1 = W_gat[:, :D].T
    wg2 = W_gat[:, D:].T
    att_src2 = att_src.reshape(1, O)
    att_dst2 = att_dst.reshape(1, O)

    h, aa = _tc1(x_lin, x_conv, wg1, wg2, att_src2, att_dst2)

    acc, den = _sc_call(h, aa, src_t, dst_t)

    wlt = W_lin.T
    out_lin, out_conv = _tc2(acc, den, x_lin, wlt,
                             b_lin.reshape(1, O), b_gat.reshape(1, O))
    return (out_lin, out_conv)
